# baseline (speedup 1.0000x reference)
import functools

import jax
import jax.numpy as jnp
from jax import lax
from jax.experimental import pallas as pl
from jax.experimental.pallas import tpu as pltpu
from jax.experimental.pallas import tpu_sc as plsc

_NC = 2
_NS = 16
_NW = _NC * _NS
_CHUNK = 128
_GRP = 8
_DEG_G = 8
_DEG_W = 16
_RBLK = 80
_PAD = 96


def _deg_count(dst2d, n_nodes):
  n_groups = dst2d.shape[0] // _DEG_G
  iters = (n_groups + _NW - 1) // _NW
  n_blk = n_nodes // _RBLK
  blk_iters = (n_blk + _NS - 1) // _NS
  mesh = plsc.VectorSubcoreMesh(core_axis_name="c", subcore_axis_name="s")
  ones = jnp.ones((_CHUNK, _DEG_W), jnp.float32)
  zeros = jnp.zeros((_RBLK, _DEG_W), jnp.float32)

  @functools.partial(
      pl.kernel,
      out_type=jax.ShapeDtypeStruct((_NC, n_nodes, _DEG_W), jnp.float32),
      mesh=mesh,
      compiler_params=pltpu.CompilerParams(use_tc_tiling_on_sc=False),
      scratch_types=[
          pltpu.VMEM((_DEG_G, _CHUNK), jnp.int32),
          pltpu.VMEM((_CHUNK, _DEG_W), jnp.float32),
          pltpu.VMEM((_RBLK, _DEG_W), jnp.float32),
          pltpu.VMEM_SHARED((n_nodes + _PAD, _DEG_W), jnp.float32),
          pltpu.SemaphoreType.DMA,
      ],
  )
  def k(dst_h, ones_h, z_h, out_h, dst_v, ones_v, stage_v, acc, sem):
    cid = lax.axis_index("c")
    sid = lax.axis_index("s")
    wid = sid * _NC + cid
    pltpu.sync_copy(ones_h, ones_v)

    def zbody(i, carry):
      b = sid + i * _NS

      @pl.when(b < n_blk)
      def _():
        pltpu.sync_copy(z_h, acc.at[pl.ds(b * _RBLK, _RBLK), :])

      return carry

    lax.fori_loop(0, blk_iters, zbody, 0)
    plsc.subcore_barrier()

    def body(i, carry):
      g = wid + i * _NW

      @pl.when(g < n_groups)
      def _():
        pltpu.sync_copy(dst_h.at[pl.ds(g * _DEG_G, _DEG_G), :], dst_v)
        descs = [
            pltpu.async_copy(ones_v, acc.at[dst_v.at[j]], sem, add=True)
            for j in range(_DEG_G)
        ]
        for d_ in descs:
          d_.wait()

      return carry

    lax.fori_loop(0, iters, body, 0)
    plsc.subcore_barrier()

    def obody(i, carry):
      b = sid + i * _NS

      @pl.when(b < n_blk)
      def _():
        pltpu.sync_copy(acc.at[pl.ds(b * _RBLK, _RBLK), :], stage_v)
        pltpu.sync_copy(stage_v, out_h.at[cid, pl.ds(b * _RBLK, _RBLK), :])

      return carry

    lax.fori_loop(0, blk_iters, obody, 0)

  return k(dst2d, ones, zeros)


def _edge_agg(src2d, dst2d, table, n_nodes, nbuf, grp):
  d = table.shape[1]
  n_groups = src2d.shape[0] // grp
  iters = (n_groups + _NW - 1) // _NW
  n_blk = n_nodes // _RBLK
  blk_iters = (n_blk + _NS - 1) // _NS
  mesh = plsc.VectorSubcoreMesh(core_axis_name="c", subcore_axis_name="s")
  zeros = jnp.zeros((_RBLK, d), jnp.float32)

  @functools.partial(
      pl.kernel,
      out_type=jax.ShapeDtypeStruct((_NC, n_nodes, d), jnp.float32),
      mesh=mesh,
      compiler_params=pltpu.CompilerParams(use_tc_tiling_on_sc=False),
      scratch_types=(
          [pltpu.VMEM((grp, _CHUNK), jnp.int32),
           pltpu.VMEM((grp, _CHUNK), jnp.int32)]
          + [pltpu.VMEM((_CHUNK, d), jnp.float32) for _ in range(nbuf)]
          + [pltpu.VMEM_SHARED((n_nodes + _PAD, d), jnp.float32)]
          + [pltpu.SemaphoreType.DMA for _ in range(nbuf)]
      ),
  )
  def k(src_h, dst_h, tab_h, z_h, out_h, src_v, dst_v, *rest):
    rows = rest[:nbuf]
    acc = rest[nbuf]
    sems = rest[nbuf + 1:]
    cid = lax.axis_index("c")
    sid = lax.axis_index("s")
    wid = sid * _NC + cid

    def zbody(i, carry):
      b = sid + i * _NS

      @pl.when(b < n_blk)
      def _():
        pltpu.sync_copy(z_h, acc.at[pl.ds(b * _RBLK, _RBLK), :])

      return carry

    lax.fori_loop(0, blk_iters, zbody, 0)
    plsc.subcore_barrier()

    def body(i, carry):
      g = wid + i * _NW

      @pl.when(g < n_groups)
      def _():
        pltpu.sync_copy(src_h.at[pl.ds(g * grp, grp), :], src_v)
        pltpu.sync_copy(dst_h.at[pl.ds(g * grp, grp), :], dst_v)
        descs = [None] * grp
        for p in range(nbuf - 1):
          descs[p] = pltpu.async_copy(
              tab_h.at[src_v.at[p]], rows[p % nbuf], sems[p % nbuf])
        for j in range(grp):
          nx = j + nbuf - 1
          if nx < grp:
            descs[nx] = pltpu.async_copy(
                tab_h.at[src_v.at[nx]], rows[nx % nbuf], sems[nx % nbuf])
          descs[j].wait()
          pltpu.sync_copy(rows[j % nbuf], acc.at[dst_v.at[j]], add=True)

      return carry

    lax.fori_loop(0, iters, body, 0)
    plsc.subcore_barrier()

    def obody(i, carry):
      b = sid + i * _NS

      @pl.when(b < n_blk)
      def _():
        pltpu.sync_copy(acc.at[pl.ds(b * _RBLK, _RBLK), :],
                        rows[0].at[pl.ds(0, _RBLK), :])
        pltpu.sync_copy(rows[0].at[pl.ds(0, _RBLK), :],
                        out_h.at[cid, pl.ds(b * _RBLK, _RBLK), :])

      return carry

    lax.fori_loop(0, blk_iters, obody, 0)

  return k(src2d, dst2d, table, zeros)


def _tc_matmul(x, w):
  def body(x_ref, w_ref, o_ref):
    o_ref[...] = jnp.dot(x_ref[...], w_ref[...],
                         preferred_element_type=jnp.float32)

  return pl.pallas_call(
      body,
      out_shape=jax.ShapeDtypeStruct((x.shape[0], w.shape[1]), jnp.float32),
  )(x, w)


def _tc_dinv_scale(degp, xw):
  n = xw.shape[0]

  def body(d_ref, xw_ref, dinv_ref, y_ref):
    dsum = d_ref[0] + d_ref[1]
    deg = dsum[:, 0:1] + 1.0
    dinv = lax.rsqrt(deg)
    dinv_ref[...] = dinv
    y_ref[...] = xw_ref[...] * dinv

  return pl.pallas_call(
      body,
      out_shape=(
          jax.ShapeDtypeStruct((n, 1), jnp.float32),
          jax.ShapeDtypeStruct(xw.shape, jnp.float32),
      ),
  )(degp, xw)


def _tc_mid(accp, y1, dinv, b1, w2):
  n = y1.shape[0]

  def body(a_ref, y_ref, di_ref, b_ref, w_ref, o_ref):
    di = di_ref[...]
    s = a_ref[0] + a_ref[1] + y_ref[...]
    h = jnp.maximum(di * s + b_ref[...], 0.0)
    o_ref[...] = di * jnp.dot(h, w_ref[...],
                              preferred_element_type=jnp.float32)

  return pl.pallas_call(
      body,
      out_shape=jax.ShapeDtypeStruct((n, w2.shape[1]), jnp.float32),
  )(accp, y1, dinv, b1, w2)


def _tc_final(accp, y2, dinv, b2):
  def body(a_ref, y_ref, di_ref, b_ref, o_ref):
    s = a_ref[0] + a_ref[1] + y_ref[...]
    o_ref[...] = di_ref[...] * s + b_ref[...]

  return pl.pallas_call(
      body,
      out_shape=jax.ShapeDtypeStruct(y2.shape, jnp.float32),
  )(accp, y2, dinv, b2)


def _pad_edges(src, dst, n_nodes, chunk, n_rows):
  n_edges = src.shape[0]
  n_pad = n_rows * chunk - n_edges
  pad_dst = n_nodes + (jnp.arange(n_pad, dtype=jnp.int32) % _PAD)
  src_p = jnp.concatenate([src, jnp.zeros((n_pad,), jnp.int32)])
  dst_p = jnp.concatenate([dst, pad_dst])
  return src_p.reshape(-1, chunk), dst_p.reshape(-1, chunk)


def kernel(x, edge_index, W1, b1, W2, b2):
  n = x.shape[0]
  n_edges = edge_index.shape[1]
  src = edge_index[0].astype(jnp.int32)
  dst = edge_index[1].astype(jnp.int32)

  n_rows = -(-n_edges // _CHUNK)
  n_rows += (-n_rows) % 16
  srcp, dstp = _pad_edges(src, dst, n, _CHUNK, n_rows)

  degp = _deg_count(dstp, n)
  xw1 = _tc_matmul(x, W1)
  dinv, y1 = _tc_dinv_scale(degp, xw1)
  acc1 = _edge_agg(srcp, dstp, y1, n, 3, 8)
  y2 = _tc_mid(acc1, y1, dinv, b1.reshape(1, -1), W2)
  acc2 = _edge_agg(srcp, dstp, y2, n, 2, 8)
  out = _tc_final(acc2, y2, dinv, b2.reshape(1, -1))
  return out


# speedup vs baseline: 1.0018x; 1.0018x over previous
import functools

import jax
import jax.numpy as jnp
from jax import lax
from jax.experimental import pallas as pl
from jax.experimental.pallas import tpu as pltpu
from jax.experimental.pallas import tpu_sc as plsc

_NC = 2
_NS = 16
_NW = _NC * _NS
_CHUNK = 128
_GRP = 8
_DEG_G = 8
_DEG_W = 16
_RBLK = 80
_PAD = 128


def _deg_count(dst2d, n_nodes):
  n_groups = dst2d.shape[0] // _DEG_G
  iters = (n_groups + _NW - 1) // _NW
  n_blk = n_nodes // _RBLK
  blk_iters = (n_blk + _NS - 1) // _NS
  mesh = plsc.VectorSubcoreMesh(core_axis_name="c", subcore_axis_name="s")
  ones = jnp.ones((_CHUNK, _DEG_W), jnp.float32)
  zeros = jnp.zeros((_RBLK, _DEG_W), jnp.float32)

  @functools.partial(
      pl.kernel,
      out_type=jax.ShapeDtypeStruct((_NC, n_nodes, _DEG_W), jnp.float32),
      mesh=mesh,
      compiler_params=pltpu.CompilerParams(use_tc_tiling_on_sc=False),
      scratch_types=[
          pltpu.VMEM((_DEG_G, _CHUNK), jnp.int32),
          pltpu.VMEM((_CHUNK, _DEG_W), jnp.float32),
          pltpu.VMEM((_RBLK, _DEG_W), jnp.float32),
          pltpu.VMEM_SHARED((n_nodes + _PAD, _DEG_W), jnp.float32),
          pltpu.SemaphoreType.DMA,
      ],
  )
  def k(dst_h, ones_h, z_h, out_h, dst_v, ones_v, stage_v, acc, sem):
    cid = lax.axis_index("c")
    sid = lax.axis_index("s")
    wid = sid * _NC + cid
    pltpu.sync_copy(ones_h, ones_v)

    def zbody(i, carry):
      b = sid + i * _NS

      @pl.when(b < n_blk)
      def _():
        pltpu.sync_copy(z_h, acc.at[pl.ds(b * _RBLK, _RBLK), :])

      return carry

    lax.fori_loop(0, blk_iters, zbody, 0)
    plsc.subcore_barrier()

    def body(i, carry):
      g = wid + i * _NW

      @pl.when(g < n_groups)
      def _():
        pltpu.sync_copy(dst_h.at[pl.ds(g * _DEG_G, _DEG_G), :], dst_v)
        descs = [
            pltpu.async_copy(ones_v, acc.at[dst_v.at[j]], sem, add=True)
            for j in range(_DEG_G)
        ]
        for d_ in descs:
          d_.wait()

      return carry

    lax.fori_loop(0, iters, body, 0)
    plsc.subcore_barrier()

    def obody(i, carry):
      b = sid + i * _NS

      @pl.when(b < n_blk)
      def _():
        pltpu.sync_copy(acc.at[pl.ds(b * _RBLK, _RBLK), :], stage_v)
        pltpu.sync_copy(stage_v, out_h.at[cid, pl.ds(b * _RBLK, _RBLK), :])

      return carry

    lax.fori_loop(0, blk_iters, obody, 0)

  return k(dst2d, ones, zeros)


def _edge_agg(src2d, dst2d, table, n_nodes, nbuf, grp):
  d = table.shape[1]
  n_groups = src2d.shape[0] // grp
  iters = (n_groups + _NW - 1) // _NW
  n_blk = n_nodes // _RBLK
  blk_iters = (n_blk + _NS - 1) // _NS
  mesh = plsc.VectorSubcoreMesh(core_axis_name="c", subcore_axis_name="s")
  zeros = jnp.zeros((_RBLK, d), jnp.float32)

  @functools.partial(
      pl.kernel,
      out_type=jax.ShapeDtypeStruct((_NC, n_nodes, d), jnp.float32),
      mesh=mesh,
      compiler_params=pltpu.CompilerParams(use_tc_tiling_on_sc=False),
      scratch_types=(
          [pltpu.VMEM((grp, _CHUNK), jnp.int32),
           pltpu.VMEM((grp, _CHUNK), jnp.int32)]
          + [pltpu.VMEM((_CHUNK, d), jnp.float32) for _ in range(nbuf)]
          + [pltpu.VMEM((_RBLK, d), jnp.float32),
             pltpu.VMEM_SHARED((n_nodes + _PAD, d), jnp.float32)]
          + [pltpu.SemaphoreType.DMA for _ in range(nbuf)]
      ),
  )
  def k(src_h, dst_h, tab_h, z_h, out_h, src_v, dst_v, *rest):
    rows = rest[:nbuf]
    stage_v = rest[nbuf]
    acc = rest[nbuf + 1]
    sems = rest[nbuf + 2:]
    cid = lax.axis_index("c")
    sid = lax.axis_index("s")
    wid = sid * _NC + cid

    def zbody(i, carry):
      b = sid + i * _NS

      @pl.when(b < n_blk)
      def _():
        pltpu.sync_copy(z_h, acc.at[pl.ds(b * _RBLK, _RBLK), :])

      return carry

    lax.fori_loop(0, blk_iters, zbody, 0)
    plsc.subcore_barrier()

    def body(i, carry):
      g = wid + i * _NW

      @pl.when(g < n_groups)
      def _():
        pltpu.sync_copy(src_h.at[pl.ds(g * grp, grp), :], src_v)
        pltpu.sync_copy(dst_h.at[pl.ds(g * grp, grp), :], dst_v)
        descs = [None] * grp
        for p in range(nbuf - 1):
          descs[p] = pltpu.async_copy(
              tab_h.at[src_v.at[p]], rows[p % nbuf], sems[p % nbuf])
        for j in range(grp):
          nx = j + nbuf - 1
          if nx < grp:
            descs[nx] = pltpu.async_copy(
                tab_h.at[src_v.at[nx]], rows[nx % nbuf], sems[nx % nbuf])
          descs[j].wait()
          pltpu.sync_copy(rows[j % nbuf], acc.at[dst_v.at[j]], add=True)

      return carry

    lax.fori_loop(0, iters, body, 0)
    plsc.subcore_barrier()

    def obody(i, carry):
      b = sid + i * _NS

      @pl.when(b < n_blk)
      def _():
        pltpu.sync_copy(acc.at[pl.ds(b * _RBLK, _RBLK), :], stage_v)
        pltpu.sync_copy(stage_v, out_h.at[cid, pl.ds(b * _RBLK, _RBLK), :])

      return carry

    lax.fori_loop(0, blk_iters, obody, 0)

  return k(src2d, dst2d, table, zeros)


def _tc_matmul(x, w):
  def body(x_ref, w_ref, o_ref):
    o_ref[...] = jnp.dot(x_ref[...], w_ref[...],
                         preferred_element_type=jnp.float32)

  return pl.pallas_call(
      body,
      out_shape=jax.ShapeDtypeStruct((x.shape[0], w.shape[1]), jnp.float32),
  )(x, w)


def _tc_dinv_scale(degp, xw):
  n = xw.shape[0]

  def body(d_ref, xw_ref, dinv_ref, y_ref):
    dsum = d_ref[0] + d_ref[1]
    deg = dsum[:, 0:1] + 1.0
    dinv = lax.rsqrt(deg)
    dinv_ref[...] = dinv
    y_ref[...] = xw_ref[...] * dinv

  return pl.pallas_call(
      body,
      out_shape=(
          jax.ShapeDtypeStruct((n, 1), jnp.float32),
          jax.ShapeDtypeStruct(xw.shape, jnp.float32),
      ),
  )(degp, xw)


def _tc_mid(accp, y1, dinv, b1, w2):
  n = y1.shape[0]

  def body(a_ref, y_ref, di_ref, b_ref, w_ref, o_ref):
    di = di_ref[...]
    s = a_ref[0] + a_ref[1] + y_ref[...]
    h = jnp.maximum(di * s + b_ref[...], 0.0)
    o_ref[...] = di * jnp.dot(h, w_ref[...],
                              preferred_element_type=jnp.float32)

  return pl.pallas_call(
      body,
      out_shape=jax.ShapeDtypeStruct((n, w2.shape[1]), jnp.float32),
  )(accp, y1, dinv, b1, w2)


def _tc_final(accp, y2, dinv, b2):
  def body(a_ref, y_ref, di_ref, b_ref, o_ref):
    s = a_ref[0] + a_ref[1] + y_ref[...]
    o_ref[...] = di_ref[...] * s + b_ref[...]

  return pl.pallas_call(
      body,
      out_shape=jax.ShapeDtypeStruct(y2.shape, jnp.float32),
  )(accp, y2, dinv, b2)


def _pad_edges(src, dst, n_nodes, chunk, n_rows):
  n_edges = src.shape[0]
  n_pad = n_rows * chunk - n_edges
  pad_dst = n_nodes + (jnp.arange(n_pad, dtype=jnp.int32) % _PAD)
  src_p = jnp.concatenate([src, jnp.zeros((n_pad,), jnp.int32)])
  dst_p = jnp.concatenate([dst, pad_dst])
  return src_p.reshape(-1, chunk), dst_p.reshape(-1, chunk)


def kernel(x, edge_index, W1, b1, W2, b2):
  n = x.shape[0]
  n_edges = edge_index.shape[1]
  src = edge_index[0].astype(jnp.int32)
  dst = edge_index[1].astype(jnp.int32)

  n_rows = -(-n_edges // _CHUNK)
  n_rows += (-n_rows) % 16
  srcp, dstp = _pad_edges(src, dst, n, _CHUNK, n_rows)

  degp = _deg_count(dstp, n)
  xw1 = _tc_matmul(x, W1)
  dinv, y1 = _tc_dinv_scale(degp, xw1)
  acc1 = _edge_agg(srcp, dstp, y1, n, 3, 8)
  y2 = _tc_mid(acc1, y1, dinv, b1.reshape(1, -1), W2)
  acc2 = _edge_agg(srcp, dstp, y2, n, 2, 8)
  out = _tc_final(acc2, y2, dinv, b2.reshape(1, -1))
  return out


# speedup vs baseline: 1.1646x; 1.1624x over previous
import functools

import jax
import jax.numpy as jnp
from jax import lax
from jax.experimental import pallas as pl
from jax.experimental.pallas import tpu as pltpu
from jax.experimental.pallas import tpu_sc as plsc

_NC = 2
_NS = 16
_NW = _NC * _NS
_CHUNK = 128
_GRP = 8
_DEG_G = 8
_DEG_W = 16
_RBLK = 80
_PAD = 128


def _deg_count(dst2d, n_nodes):
  n_groups = dst2d.shape[0] // _DEG_G
  iters = (n_groups + _NW - 1) // _NW
  n_blk = n_nodes // _RBLK
  blk_iters = (n_blk + _NS - 1) // _NS
  mesh = plsc.VectorSubcoreMesh(core_axis_name="c", subcore_axis_name="s")
  ones = jnp.ones((_CHUNK, _DEG_W), jnp.float32)
  zeros = jnp.zeros((_RBLK, _DEG_W), jnp.float32)

  @functools.partial(
      pl.kernel,
      out_type=jax.ShapeDtypeStruct((_NC, n_nodes, _DEG_W), jnp.float32),
      mesh=mesh,
      compiler_params=pltpu.CompilerParams(use_tc_tiling_on_sc=False),
      scratch_types=[
          pltpu.VMEM((_DEG_G, _CHUNK), jnp.int32),
          pltpu.VMEM((_CHUNK, _DEG_W), jnp.float32),
          pltpu.VMEM((_RBLK, _DEG_W), jnp.float32),
          pltpu.VMEM_SHARED((n_nodes + _PAD, _DEG_W), jnp.float32),
          pltpu.SemaphoreType.DMA,
      ],
  )
  def k(dst_h, ones_h, z_h, out_h, dst_v, ones_v, stage_v, acc, sem):
    cid = lax.axis_index("c")
    sid = lax.axis_index("s")
    wid = sid * _NC + cid
    pltpu.sync_copy(ones_h, ones_v)

    def zbody(i, carry):
      b = sid + i * _NS

      @pl.when(b < n_blk)
      def _():
        pltpu.sync_copy(z_h, acc.at[pl.ds(b * _RBLK, _RBLK), :])

      return carry

    lax.fori_loop(0, blk_iters, zbody, 0)
    plsc.subcore_barrier()

    def body(i, carry):
      g = wid + i * _NW

      @pl.when(g < n_groups)
      def _():
        pltpu.sync_copy(dst_h.at[pl.ds(g * _DEG_G, _DEG_G), :], dst_v)
        descs = [
            pltpu.async_copy(ones_v, acc.at[dst_v.at[j]], sem, add=True)
            for j in range(_DEG_G)
        ]
        for d_ in descs:
          d_.wait()

      return carry

    lax.fori_loop(0, iters, body, 0)
    plsc.subcore_barrier()

    def obody(i, carry):
      b = sid + i * _NS

      @pl.when(b < n_blk)
      def _():
        pltpu.sync_copy(acc.at[pl.ds(b * _RBLK, _RBLK), :], stage_v)
        pltpu.sync_copy(stage_v, out_h.at[cid, pl.ds(b * _RBLK, _RBLK), :])

      return carry

    lax.fori_loop(0, blk_iters, obody, 0)

  return k(dst2d, ones, zeros)


def _edge_agg(src2d, dst2d, table, n_nodes, nbuf, grp):
  d = table.shape[1]
  n_groups = src2d.shape[0] // grp
  iters = (n_groups + _NW - 1) // _NW
  n_blk = n_nodes // _RBLK
  blk_iters = (n_blk + _NS - 1) // _NS
  mesh = plsc.VectorSubcoreMesh(core_axis_name="c", subcore_axis_name="s")
  zeros = jnp.zeros((_RBLK, d), jnp.float32)

  @functools.partial(
      pl.kernel,
      out_type=jax.ShapeDtypeStruct((_NC, n_nodes, d), jnp.float32),
      mesh=mesh,
      compiler_params=pltpu.CompilerParams(use_tc_tiling_on_sc=False),
      scratch_types=(
          [pltpu.VMEM((grp, _CHUNK), jnp.int32),
           pltpu.VMEM((grp, _CHUNK), jnp.int32)]
          + [pltpu.VMEM((_CHUNK, d), jnp.float32) for _ in range(nbuf)]
          + [pltpu.VMEM((_RBLK, d), jnp.float32),
             pltpu.VMEM_SHARED((n_nodes + _PAD, d), jnp.float32)]
          + [pltpu.SemaphoreType.DMA for _ in range(nbuf)]
      ),
  )
  def k(src_h, dst_h, tab_h, z_h, out_h, src_v, dst_v, *rest):
    rows = rest[:nbuf]
    stage_v = rest[nbuf]
    acc = rest[nbuf + 1]
    sems = rest[nbuf + 2:]
    cid = lax.axis_index("c")
    sid = lax.axis_index("s")
    wid = sid * _NC + cid

    def zbody(i, carry):
      b = sid + i * _NS

      @pl.when(b < n_blk)
      def _():
        pltpu.sync_copy(z_h, acc.at[pl.ds(b * _RBLK, _RBLK), :])

      return carry

    lax.fori_loop(0, blk_iters, zbody, 0)
    plsc.subcore_barrier()

    def body(i, carry):
      g = wid + i * _NW

      @pl.when(g < n_groups)
      def _():
        pltpu.sync_copy(src_h.at[pl.ds(g * grp, grp), :], src_v)
        pltpu.sync_copy(dst_h.at[pl.ds(g * grp, grp), :], dst_v)
        descs = [None] * grp
        for p in range(nbuf - 1):
          descs[p] = pltpu.async_copy(
              tab_h.at[src_v.at[p]], rows[p % nbuf], sems[p % nbuf])
        for j in range(grp):
          nx = j + nbuf - 1
          if nx < grp:
            descs[nx] = pltpu.async_copy(
                tab_h.at[src_v.at[nx]], rows[nx % nbuf], sems[nx % nbuf])
          descs[j].wait()
          pltpu.sync_copy(rows[j % nbuf], acc.at[dst_v.at[j]], add=True)

      return carry

    lax.fori_loop(0, iters, body, 0)
    plsc.subcore_barrier()

    def obody(i, carry):
      b = sid + i * _NS

      @pl.when(b < n_blk)
      def _():
        pltpu.sync_copy(acc.at[pl.ds(b * _RBLK, _RBLK), :], stage_v)
        pltpu.sync_copy(stage_v, out_h.at[cid, pl.ds(b * _RBLK, _RBLK), :])

      return carry

    lax.fori_loop(0, blk_iters, obody, 0)

  return k(src2d, dst2d, table, zeros)


def _tc_matmul(x, w):
  def body(x_ref, w_ref, o_ref):
    o_ref[...] = jnp.dot(x_ref[...], w_ref[...],
                         preferred_element_type=jnp.float32)

  return pl.pallas_call(
      body,
      out_shape=jax.ShapeDtypeStruct((x.shape[0], w.shape[1]), jnp.float32),
  )(x, w)


def _tc_dinv_scale(degp, xw):
  n = xw.shape[0]

  def body(d_ref, xw_ref, dinv_ref, y_ref):
    dsum = d_ref[0] + d_ref[1]
    deg = dsum[:, 0:1] + 1.0
    dinv = lax.rsqrt(deg)
    dinv_ref[...] = dinv
    y_ref[...] = xw_ref[...] * dinv

  return pl.pallas_call(
      body,
      out_shape=(
          jax.ShapeDtypeStruct((n, 1), jnp.float32),
          jax.ShapeDtypeStruct(xw.shape, jnp.float32),
      ),
  )(degp, xw)


def _tc_mid(accp, y1, dinv, b1, w2):
  n = y1.shape[0]

  def body(a_ref, y_ref, di_ref, b_ref, w_ref, o_ref):
    di = di_ref[...]
    s = a_ref[0] + a_ref[1] + y_ref[...]
    h = jnp.maximum(di * s + b_ref[...], 0.0)
    o_ref[...] = di * jnp.dot(h, w_ref[...],
                              preferred_element_type=jnp.float32)

  return pl.pallas_call(
      body,
      out_shape=jax.ShapeDtypeStruct((n, w2.shape[1]), jnp.float32),
  )(accp, y1, dinv, b1, w2)


def _tc_final(accp, y2, dinv, b2):
  def body(a_ref, y_ref, di_ref, b_ref, o_ref):
    s = a_ref[0] + a_ref[1] + y_ref[...]
    o_ref[...] = di_ref[...] * s + b_ref[...]

  return pl.pallas_call(
      body,
      out_shape=jax.ShapeDtypeStruct(y2.shape, jnp.float32),
  )(accp, y2, dinv, b2)


def _pad_edges(src, dst, n_nodes, chunk, n_rows):
  n_edges = src.shape[0]
  n_pad = n_rows * chunk - n_edges
  pad_dst = n_nodes + (jnp.arange(n_pad, dtype=jnp.int32) % _PAD)
  src_p = jnp.concatenate([src, jnp.zeros((n_pad,), jnp.int32)])
  dst_p = jnp.concatenate([dst, pad_dst])
  return src_p.reshape(-1, chunk), dst_p.reshape(-1, chunk)


def kernel(x, edge_index, W1, b1, W2, b2):
  n = x.shape[0]
  n_edges = edge_index.shape[1]
  src = edge_index[0].astype(jnp.int32)
  dst = edge_index[1].astype(jnp.int32)

  n_rows = -(-n_edges // _CHUNK)
  n_rows += (-n_rows) % _GRP
  srcp, dstp = _pad_edges(src, dst, n, _CHUNK, n_rows)

  degp = _deg_count(dstp, n)
  xw1 = _tc_matmul(x, W1)
  dinv, y1 = _tc_dinv_scale(degp, xw1)
  acc1 = _edge_agg(srcp, dstp, y1, n, 3, 8)
  y2 = _tc_mid(acc1, y1, dinv, b1.reshape(1, -1), W2)
  acc2 = _edge_agg(srcp, dstp, y2, n, 2, 8)
  out = _tc_final(acc2, y2, dinv, b2.reshape(1, -1))
  return out


# speedup vs baseline: 1.1708x; 1.0054x over previous
import functools

import jax
import jax.numpy as jnp
from jax import lax
from jax.experimental import pallas as pl
from jax.experimental.pallas import tpu as pltpu
from jax.experimental.pallas import tpu_sc as plsc

_NC = 2
_NS = 16
_NW = _NC * _NS
_CHUNK = 128
_GRP = 8
_DEG_G = 8
_DEG_W = 16
_RBLK = 80
_PAD = 128


def _deg_count(dst2d, n_nodes):
  n_groups = dst2d.shape[0] // _DEG_G
  iters = (n_groups + _NW - 1) // _NW
  n_blk = n_nodes // _RBLK
  blk_iters = (n_blk + _NS - 1) // _NS
  mesh = plsc.VectorSubcoreMesh(core_axis_name="c", subcore_axis_name="s")
  ones = jnp.ones((_CHUNK, _DEG_W), jnp.float32)
  zeros = jnp.zeros((_RBLK, _DEG_W), jnp.float32)

  @functools.partial(
      pl.kernel,
      out_type=jax.ShapeDtypeStruct((_NC, n_nodes, _DEG_W), jnp.float32),
      mesh=mesh,
      compiler_params=pltpu.CompilerParams(use_tc_tiling_on_sc=False),
      scratch_types=[
          pltpu.VMEM((_DEG_G, _CHUNK), jnp.int32),
          pltpu.VMEM((_CHUNK, _DEG_W), jnp.float32),
          pltpu.VMEM((_RBLK, _DEG_W), jnp.float32),
          pltpu.VMEM_SHARED((n_nodes + _PAD, _DEG_W), jnp.float32),
          pltpu.SemaphoreType.DMA,
      ],
  )
  def k(dst_h, ones_h, z_h, out_h, dst_v, ones_v, stage_v, acc, sem):
    cid = lax.axis_index("c")
    sid = lax.axis_index("s")
    wid = sid * _NC + cid
    pltpu.sync_copy(ones_h, ones_v)

    def zbody(i, carry):
      b = sid + i * _NS

      @pl.when(b < n_blk)
      def _():
        pltpu.sync_copy(z_h, acc.at[pl.ds(b * _RBLK, _RBLK), :])

      return carry

    lax.fori_loop(0, blk_iters, zbody, 0)
    plsc.subcore_barrier()

    def body(i, carry):
      g = wid + i * _NW

      @pl.when(g < n_groups)
      def _():
        pltpu.sync_copy(dst_h.at[pl.ds(g * _DEG_G, _DEG_G), :], dst_v)
        descs = [
            pltpu.async_copy(ones_v, acc.at[dst_v.at[j]], sem, add=True)
            for j in range(_DEG_G)
        ]
        for d_ in descs:
          d_.wait()

      return carry

    lax.fori_loop(0, iters, body, 0)
    plsc.subcore_barrier()

    def obody(i, carry):
      b = sid + i * _NS

      @pl.when(b < n_blk)
      def _():
        pltpu.sync_copy(acc.at[pl.ds(b * _RBLK, _RBLK), :], stage_v)
        pltpu.sync_copy(stage_v, out_h.at[cid, pl.ds(b * _RBLK, _RBLK), :])

      return carry

    lax.fori_loop(0, blk_iters, obody, 0)

  return k(dst2d, ones, zeros)


def _edge_agg(src2d, dst2d, table, n_nodes, nbuf, grp):
  d = table.shape[1]
  n_groups = src2d.shape[0] // grp
  iters = (n_groups + _NW - 1) // _NW
  n_blk = n_nodes // _RBLK
  blk_iters = (n_blk + _NS - 1) // _NS
  mesh = plsc.VectorSubcoreMesh(core_axis_name="c", subcore_axis_name="s")
  zeros = jnp.zeros((_RBLK, d), jnp.float32)

  @functools.partial(
      pl.kernel,
      out_type=jax.ShapeDtypeStruct((_NC, n_nodes, d), jnp.float32),
      mesh=mesh,
      compiler_params=pltpu.CompilerParams(use_tc_tiling_on_sc=False),
      scratch_types=(
          [pltpu.VMEM((grp, _CHUNK), jnp.int32),
           pltpu.VMEM((grp, _CHUNK), jnp.int32)]
          + [pltpu.VMEM((_CHUNK, d), jnp.float32) for _ in range(nbuf)]
          + [pltpu.VMEM((_RBLK, d), jnp.float32),
             pltpu.VMEM_SHARED((n_nodes + _PAD, d), jnp.float32)]
          + [pltpu.SemaphoreType.DMA for _ in range(nbuf)]
      ),
  )
  def k(src_h, dst_h, tab_h, z_h, out_h, src_v, dst_v, *rest):
    rows = rest[:nbuf]
    stage_v = rest[nbuf]
    acc = rest[nbuf + 1]
    sems = rest[nbuf + 2:]
    cid = lax.axis_index("c")
    sid = lax.axis_index("s")
    wid = sid * _NC + cid

    def zbody(i, carry):
      b = sid + i * _NS

      @pl.when(b < n_blk)
      def _():
        pltpu.sync_copy(z_h, acc.at[pl.ds(b * _RBLK, _RBLK), :])

      return carry

    lax.fori_loop(0, blk_iters, zbody, 0)
    plsc.subcore_barrier()

    def body(i, carry):
      g = wid + i * _NW

      @pl.when(g < n_groups)
      def _():
        pltpu.sync_copy(src_h.at[pl.ds(g * grp, grp), :], src_v)
        pltpu.sync_copy(dst_h.at[pl.ds(g * grp, grp), :], dst_v)
        descs = [None] * grp
        for p in range(nbuf - 1):
          descs[p] = pltpu.async_copy(
              tab_h.at[src_v.at[p]], rows[p % nbuf], sems[p % nbuf])
        for j in range(grp):
          nx = j + nbuf - 1
          if nx < grp:
            descs[nx] = pltpu.async_copy(
                tab_h.at[src_v.at[nx]], rows[nx % nbuf], sems[nx % nbuf])
          descs[j].wait()
          pltpu.sync_copy(rows[j % nbuf], acc.at[dst_v.at[j]], add=True)

      return carry

    lax.fori_loop(0, iters, body, 0)
    plsc.subcore_barrier()

    def obody(i, carry):
      b = sid + i * _NS

      @pl.when(b < n_blk)
      def _():
        pltpu.sync_copy(acc.at[pl.ds(b * _RBLK, _RBLK), :], stage_v)
        pltpu.sync_copy(stage_v, out_h.at[cid, pl.ds(b * _RBLK, _RBLK), :])

      return carry

    lax.fori_loop(0, blk_iters, obody, 0)

  return k(src2d, dst2d, table, zeros)


def _tc_dinv_scale(degp, x, w1):
  n = x.shape[0]

  def body(d_ref, x_ref, w_ref, dinv_ref, y_ref):
    dsum = d_ref[0] + d_ref[1]
    deg = dsum[:, 0:1] + 1.0
    dinv = lax.rsqrt(deg)
    dinv_ref[...] = dinv
    xw = jnp.dot(x_ref[...], w_ref[...], preferred_element_type=jnp.float32)
    y_ref[...] = xw * dinv

  return pl.pallas_call(
      body,
      out_shape=(
          jax.ShapeDtypeStruct((n, 1), jnp.float32),
          jax.ShapeDtypeStruct((n, w1.shape[1]), jnp.float32),
      ),
  )(degp, x, w1)


def _tc_mid(accp, y1, dinv, b1, w2):
  n = y1.shape[0]

  def body(a_ref, y_ref, di_ref, b_ref, w_ref, o_ref):
    di = di_ref[...]
    s = a_ref[0] + a_ref[1] + y_ref[...]
    h = jnp.maximum(di * s + b_ref[...], 0.0)
    o_ref[...] = di * jnp.dot(h, w_ref[...],
                              preferred_element_type=jnp.float32)

  return pl.pallas_call(
      body,
      out_shape=jax.ShapeDtypeStruct((n, w2.shape[1]), jnp.float32),
  )(accp, y1, dinv, b1, w2)


def _tc_final(accp, y2, dinv, b2):
  def body(a_ref, y_ref, di_ref, b_ref, o_ref):
    s = a_ref[0] + a_ref[1] + y_ref[...]
    o_ref[...] = di_ref[...] * s + b_ref[...]

  return pl.pallas_call(
      body,
      out_shape=jax.ShapeDtypeStruct(y2.shape, jnp.float32),
  )(accp, y2, dinv, b2)


def _pad_edges(src, dst, n_nodes, chunk, n_rows):
  n_edges = src.shape[0]
  n_pad = n_rows * chunk - n_edges
  pad_dst = n_nodes + (jnp.arange(n_pad, dtype=jnp.int32) % _PAD)
  src_p = jnp.concatenate([src, jnp.zeros((n_pad,), jnp.int32)])
  dst_p = jnp.concatenate([dst, pad_dst])
  return src_p.reshape(-1, chunk), dst_p.reshape(-1, chunk)


def kernel(x, edge_index, W1, b1, W2, b2):
  n = x.shape[0]
  n_edges = edge_index.shape[1]
  src = edge_index[0].astype(jnp.int32)
  dst = edge_index[1].astype(jnp.int32)

  n_rows = -(-n_edges // _CHUNK)
  n_rows += (-n_rows) % _GRP
  srcp, dstp = _pad_edges(src, dst, n, _CHUNK, n_rows)

  degp = _deg_count(dstp, n)
  dinv, y1 = _tc_dinv_scale(degp, x, W1)
  acc1 = _edge_agg(srcp, dstp, y1, n, 4, 8)
  y2 = _tc_mid(acc1, y1, dinv, b1.reshape(1, -1), W2)
  acc2 = _edge_agg(srcp, dstp, y2, n, 2, 8)
  out = _tc_final(acc2, y2, dinv, b2.reshape(1, -1))
  return out


# speedup vs baseline: 1.1740x; 1.0027x over previous
import functools

import jax
import jax.numpy as jnp
from jax import lax
from jax.experimental import pallas as pl
from jax.experimental.pallas import tpu as pltpu
from jax.experimental.pallas import tpu_sc as plsc

_NC = 2
_NS = 16
_NW = _NC * _NS
_CHUNK = 128
_GRP = 8
_DEG_G = 8
_DEG_W = 16
_RBLK = 80
_PAD = 128


def _deg_count(dst2d, n_nodes):
  n_groups = dst2d.shape[0] // _DEG_G
  iters = (n_groups + _NW - 1) // _NW
  n_blk = n_nodes // _RBLK
  blk_iters = (n_blk + _NS - 1) // _NS
  mesh = plsc.VectorSubcoreMesh(core_axis_name="c", subcore_axis_name="s")
  ones = jnp.ones((_CHUNK, _DEG_W), jnp.float32)
  zeros = jnp.zeros((_RBLK, _DEG_W), jnp.float32)

  @functools.partial(
      pl.kernel,
      out_type=jax.ShapeDtypeStruct((_NC, n_nodes, _DEG_W), jnp.float32),
      mesh=mesh,
      compiler_params=pltpu.CompilerParams(use_tc_tiling_on_sc=False),
      scratch_types=[
          pltpu.VMEM((_DEG_G, _CHUNK), jnp.int32),
          pltpu.VMEM((_CHUNK, _DEG_W), jnp.float32),
          pltpu.VMEM((_RBLK, _DEG_W), jnp.float32),
          pltpu.VMEM_SHARED((n_nodes + _PAD, _DEG_W), jnp.float32),
          pltpu.SemaphoreType.DMA,
      ],
  )
  def k(dst_h, ones_h, z_h, out_h, dst_v, ones_v, stage_v, acc, sem):
    cid = lax.axis_index("c")
    sid = lax.axis_index("s")
    wid = sid * _NC + cid
    pltpu.sync_copy(ones_h, ones_v)

    def zbody(i, carry):
      b = sid + i * _NS

      @pl.when(b < n_blk)
      def _():
        pltpu.sync_copy(z_h, acc.at[pl.ds(b * _RBLK, _RBLK), :])

      return carry

    lax.fori_loop(0, blk_iters, zbody, 0)
    plsc.subcore_barrier()

    def body(i, carry):
      g = wid + i * _NW

      @pl.when(g < n_groups)
      def _():
        pltpu.sync_copy(dst_h.at[pl.ds(g * _DEG_G, _DEG_G), :], dst_v)
        descs = [
            pltpu.async_copy(ones_v, acc.at[dst_v.at[j]], sem, add=True)
            for j in range(_DEG_G)
        ]
        for d_ in descs:
          d_.wait()

      return carry

    lax.fori_loop(0, iters, body, 0)
    plsc.subcore_barrier()

    def obody(i, carry):
      b = sid + i * _NS

      @pl.when(b < n_blk)
      def _():
        pltpu.sync_copy(acc.at[pl.ds(b * _RBLK, _RBLK), :], stage_v)
        pltpu.sync_copy(stage_v, out_h.at[cid, pl.ds(b * _RBLK, _RBLK), :])

      return carry

    lax.fori_loop(0, blk_iters, obody, 0)

  return k(dst2d, ones, zeros)


def _edge_agg(src2d, dst2d, table, n_nodes, nbuf, grp):
  d = table.shape[1]
  n_groups = src2d.shape[0] // grp
  iters = (n_groups + _NW - 1) // _NW
  n_blk = n_nodes // _RBLK
  blk_iters = (n_blk + _NS - 1) // _NS
  mesh = plsc.VectorSubcoreMesh(core_axis_name="c", subcore_axis_name="s")
  zeros = jnp.zeros((_RBLK, d), jnp.float32)

  @functools.partial(
      pl.kernel,
      out_type=jax.ShapeDtypeStruct((_NC, n_nodes, d), jnp.float32),
      mesh=mesh,
      compiler_params=pltpu.CompilerParams(use_tc_tiling_on_sc=False),
      scratch_types=(
          [pltpu.VMEM((grp, _CHUNK), jnp.int32),
           pltpu.VMEM((grp, _CHUNK), jnp.int32)]
          + [pltpu.VMEM((_CHUNK, d), jnp.float32) for _ in range(nbuf)]
          + [pltpu.VMEM((_RBLK, d), jnp.float32),
             pltpu.VMEM_SHARED((n_nodes + _PAD, d), jnp.float32)]
          + [pltpu.SemaphoreType.DMA for _ in range(2 * nbuf)]
      ),
  )
  def k(src_h, dst_h, tab_h, z_h, out_h, src_v, dst_v, *rest):
    rows = rest[:nbuf]
    stage_v = rest[nbuf]
    acc = rest[nbuf + 1]
    sems = rest[nbuf + 2:2 * nbuf + 2]
    ssems = rest[2 * nbuf + 2:]
    cid = lax.axis_index("c")
    sid = lax.axis_index("s")
    wid = sid * _NC + cid

    def zbody(i, carry):
      b = sid + i * _NS

      @pl.when(b < n_blk)
      def _():
        pltpu.sync_copy(z_h, acc.at[pl.ds(b * _RBLK, _RBLK), :])

      return carry

    lax.fori_loop(0, blk_iters, zbody, 0)
    plsc.subcore_barrier()

    def body(i, carry):
      g = wid + i * _NW

      @pl.when(g < n_groups)
      def _():
        pltpu.sync_copy(src_h.at[pl.ds(g * grp, grp), :], src_v)
        pltpu.sync_copy(dst_h.at[pl.ds(g * grp, grp), :], dst_v)
        gd = [None] * grp
        sd = [None] * grp
        for p in range(nbuf - 1):
          gd[p] = pltpu.async_copy(
              tab_h.at[src_v.at[p]], rows[p % nbuf], sems[p % nbuf])
        for j in range(grp):
          if j >= 1:
            sd[j - 1].wait()
          nx = j + nbuf - 1
          if nx < grp:
            gd[nx] = pltpu.async_copy(
                tab_h.at[src_v.at[nx]], rows[nx % nbuf], sems[nx % nbuf])
          gd[j].wait()
          sd[j] = pltpu.async_copy(rows[j % nbuf], acc.at[dst_v.at[j]],
                                   ssems[j % nbuf], add=True)
        sd[grp - 1].wait()

      return carry

    lax.fori_loop(0, iters, body, 0)
    plsc.subcore_barrier()

    def obody(i, carry):
      b = sid + i * _NS

      @pl.when(b < n_blk)
      def _():
        pltpu.sync_copy(acc.at[pl.ds(b * _RBLK, _RBLK), :], stage_v)
        pltpu.sync_copy(stage_v, out_h.at[cid, pl.ds(b * _RBLK, _RBLK), :])

      return carry

    lax.fori_loop(0, blk_iters, obody, 0)

  return k(src2d, dst2d, table, zeros)


def _tc_dinv_scale(degp, x, w1):
  n = x.shape[0]

  def body(d_ref, x_ref, w_ref, dinv_ref, y_ref):
    dsum = d_ref[0] + d_ref[1]
    deg = dsum[:, 0:1] + 1.0
    dinv = lax.rsqrt(deg)
    dinv_ref[...] = dinv
    xw = jnp.dot(x_ref[...], w_ref[...], preferred_element_type=jnp.float32)
    y_ref[...] = xw * dinv

  return pl.pallas_call(
      body,
      out_shape=(
          jax.ShapeDtypeStruct((n, 1), jnp.float32),
          jax.ShapeDtypeStruct((n, w1.shape[1]), jnp.float32),
      ),
  )(degp, x, w1)


def _tc_mid(accp, y1, dinv, b1, w2):
  n = y1.shape[0]

  def body(a_ref, y_ref, di_ref, b_ref, w_ref, o_ref):
    di = di_ref[...]
    s = a_ref[0] + a_ref[1] + y_ref[...]
    h = jnp.maximum(di * s + b_ref[...], 0.0)
    o_ref[...] = di * jnp.dot(h, w_ref[...],
                              preferred_element_type=jnp.float32)

  return pl.pallas_call(
      body,
      out_shape=jax.ShapeDtypeStruct((n, w2.shape[1]), jnp.float32),
  )(accp, y1, dinv, b1, w2)


def _tc_final(accp, y2, dinv, b2):
  def body(a_ref, y_ref, di_ref, b_ref, o_ref):
    s = a_ref[0] + a_ref[1] + y_ref[...]
    o_ref[...] = di_ref[...] * s + b_ref[...]

  return pl.pallas_call(
      body,
      out_shape=jax.ShapeDtypeStruct(y2.shape, jnp.float32),
  )(accp, y2, dinv, b2)


def _pad_edges(src, dst, n_nodes, chunk, n_rows):
  n_edges = src.shape[0]
  n_pad = n_rows * chunk - n_edges
  pad_dst = n_nodes + (jnp.arange(n_pad, dtype=jnp.int32) % _PAD)
  src_p = jnp.concatenate([src, jnp.zeros((n_pad,), jnp.int32)])
  dst_p = jnp.concatenate([dst, pad_dst])
  return src_p.reshape(-1, chunk), dst_p.reshape(-1, chunk)


def kernel(x, edge_index, W1, b1, W2, b2):
  n = x.shape[0]
  n_edges = edge_index.shape[1]
  src = edge_index[0].astype(jnp.int32)
  dst = edge_index[1].astype(jnp.int32)

  n_rows = -(-n_edges // _CHUNK)
  n_rows += (-n_rows) % _GRP
  srcp, dstp = _pad_edges(src, dst, n, _CHUNK, n_rows)

  degp = _deg_count(dstp, n)
  dinv, y1 = _tc_dinv_scale(degp, x, W1)
  acc1 = _edge_agg(srcp, dstp, y1, n, 4, 8)
  y2 = _tc_mid(acc1, y1, dinv, b1.reshape(1, -1), W2)
  acc2 = _edge_agg(srcp, dstp, y2, n, 2, 8)
  out = _tc_final(acc2, y2, dinv, b2.reshape(1, -1))
  return out


# speedup vs baseline: 1.2848x; 1.0944x over previous
import functools

import jax
import jax.numpy as jnp
from jax import lax
from jax.experimental import pallas as pl
from jax.experimental.pallas import tpu as pltpu
from jax.experimental.pallas import tpu_sc as plsc

_NC = 2
_NS = 16
_NW = _NC * _NS
_CHUNK = 128
_GRP = 8
_DEG_G = 8
_DEG_W = 16
_RBLK = 80
_PAD = 128


def _deg_count(dst2d, n_nodes):
  n_groups = dst2d.shape[0] // _DEG_G
  iters = (n_groups + _NW - 1) // _NW
  n_blk = n_nodes // _RBLK
  blk_iters = (n_blk + _NS - 1) // _NS
  mesh = plsc.VectorSubcoreMesh(core_axis_name="c", subcore_axis_name="s")
  ones = jnp.ones((_CHUNK, _DEG_W), jnp.float32)
  zeros = jnp.zeros((_RBLK, _DEG_W), jnp.float32)

  @functools.partial(
      pl.kernel,
      out_type=jax.ShapeDtypeStruct((_NC, n_nodes, _DEG_W), jnp.float32),
      mesh=mesh,
      compiler_params=pltpu.CompilerParams(use_tc_tiling_on_sc=False),
      scratch_types=[
          pltpu.VMEM((_DEG_G, _CHUNK), jnp.int32),
          pltpu.VMEM((_CHUNK, _DEG_W), jnp.float32),
          pltpu.VMEM((_RBLK, _DEG_W), jnp.float32),
          pltpu.VMEM_SHARED((n_nodes + _PAD, _DEG_W), jnp.float32),
          pltpu.SemaphoreType.DMA,
      ],
  )
  def k(dst_h, ones_h, z_h, out_h, dst_v, ones_v, stage_v, acc, sem):
    cid = lax.axis_index("c")
    sid = lax.axis_index("s")
    wid = sid * _NC + cid
    pltpu.sync_copy(ones_h, ones_v)

    def zbody(i, carry):
      b = sid + i * _NS

      @pl.when(b < n_blk)
      def _():
        pltpu.sync_copy(z_h, acc.at[pl.ds(b * _RBLK, _RBLK), :])

      return carry

    lax.fori_loop(0, blk_iters, zbody, 0)
    plsc.subcore_barrier()

    def body(i, carry):
      g = wid + i * _NW

      @pl.when(g < n_groups)
      def _():
        pltpu.sync_copy(dst_h.at[pl.ds(g * _DEG_G, _DEG_G), :], dst_v)
        descs = [
            pltpu.async_copy(ones_v, acc.at[dst_v.at[j]], sem, add=True)
            for j in range(_DEG_G)
        ]
        for d_ in descs:
          d_.wait()

      return carry

    lax.fori_loop(0, iters, body, 0)
    plsc.subcore_barrier()

    def obody(i, carry):
      b = sid + i * _NS

      @pl.when(b < n_blk)
      def _():
        pltpu.sync_copy(acc.at[pl.ds(b * _RBLK, _RBLK), :], stage_v)
        pltpu.sync_copy(stage_v, out_h.at[cid, pl.ds(b * _RBLK, _RBLK), :])

      return carry

    lax.fori_loop(0, blk_iters, obody, 0)

  return k(dst2d, ones, zeros)


def _edge_agg(src2d, dst2d, table, n_nodes, nbuf, grp):
  d = table.shape[1]
  n_groups = src2d.shape[0] // grp
  iters = (n_groups + _NW - 1) // _NW
  n_blk = n_nodes // _RBLK
  blk_iters = (n_blk + _NS - 1) // _NS
  mesh = plsc.VectorSubcoreMesh(core_axis_name="c", subcore_axis_name="s")
  zeros = jnp.zeros((_RBLK, d), jnp.float32)

  @functools.partial(
      pl.kernel,
      out_type=jax.ShapeDtypeStruct((_NC, n_nodes, d), jnp.float32),
      mesh=mesh,
      compiler_params=pltpu.CompilerParams(use_tc_tiling_on_sc=False),
      scratch_types=(
          [pltpu.VMEM((grp, _CHUNK), jnp.int32),
           pltpu.VMEM((grp, _CHUNK), jnp.int32)]
          + [pltpu.VMEM((_CHUNK, d), jnp.float32) for _ in range(nbuf)]
          + [pltpu.VMEM((_RBLK, d), jnp.float32),
             pltpu.VMEM_SHARED((n_nodes + _PAD, d), jnp.float32)]
          + [pltpu.SemaphoreType.DMA for _ in range(2 * nbuf)]
      ),
  )
  def k(src_h, dst_h, tab_h, z_h, out_h, src_v, dst_v, *rest):
    rows = rest[:nbuf]
    stage_v = rest[nbuf]
    acc = rest[nbuf + 1]
    sems = rest[nbuf + 2:2 * nbuf + 2]
    ssems = rest[2 * nbuf + 2:]
    cid = lax.axis_index("c")
    sid = lax.axis_index("s")
    wid = sid * _NC + cid

    def zbody(i, carry):
      b = sid + i * _NS

      @pl.when(b < n_blk)
      def _():
        pltpu.sync_copy(z_h, acc.at[pl.ds(b * _RBLK, _RBLK), :])

      return carry

    lax.fori_loop(0, blk_iters, zbody, 0)
    plsc.subcore_barrier()

    def body(i, carry):
      g = wid + i * _NW

      @pl.when(g < n_groups)
      def _():
        pltpu.sync_copy(src_h.at[pl.ds(g * grp, grp), :], src_v)
        pltpu.sync_copy(dst_h.at[pl.ds(g * grp, grp), :], dst_v)
        gd = [None] * grp
        sd = [None] * grp
        for p in range(nbuf - 1):
          gd[p] = pltpu.async_copy(
              tab_h.at[src_v.at[p]], rows[p % nbuf], sems[p % nbuf])
        for j in range(grp):
          if j >= 1:
            sd[j - 1].wait()
          nx = j + nbuf - 1
          if nx < grp:
            gd[nx] = pltpu.async_copy(
                tab_h.at[src_v.at[nx]], rows[nx % nbuf], sems[nx % nbuf])
          gd[j].wait()
          sd[j] = pltpu.async_copy(rows[j % nbuf], acc.at[dst_v.at[j]],
                                   ssems[j % nbuf], add=True)
        sd[grp - 1].wait()

      return carry

    lax.fori_loop(0, iters, body, 0)
    plsc.subcore_barrier()

    def obody(i, carry):
      b = sid + i * _NS

      @pl.when(b < n_blk)
      def _():
        pltpu.sync_copy(acc.at[pl.ds(b * _RBLK, _RBLK), :], stage_v)
        pltpu.sync_copy(stage_v, out_h.at[cid, pl.ds(b * _RBLK, _RBLK), :])

      return carry

    lax.fori_loop(0, blk_iters, obody, 0)

  return k(src2d, dst2d, table, zeros)


def _tc_dinv_scale(degp, x, w1):
  n = x.shape[0]

  def body(d_ref, x_ref, w_ref, dinv_ref, y_ref):
    dsum = d_ref[0] + d_ref[1]
    deg = dsum[:, 0:1] + 1.0
    dinv = lax.rsqrt(deg)
    dinv_ref[...] = dinv
    xw = jnp.dot(x_ref[...], w_ref[...], preferred_element_type=jnp.float32)
    y_ref[...] = xw * dinv

  return pl.pallas_call(
      body,
      out_shape=(
          jax.ShapeDtypeStruct((n, 1), jnp.float32),
          jax.ShapeDtypeStruct((n, w1.shape[1]), jnp.float32),
      ),
  )(degp, x, w1)


def _tc_mid(accp, y1, dinv, b1, w2):
  n = y1.shape[0]

  def body(a_ref, y_ref, di_ref, b_ref, w_ref, o_ref):
    di = di_ref[...]
    s = a_ref[0] + a_ref[1] + y_ref[...]
    h = jnp.maximum(di * s + b_ref[...], 0.0)
    o_ref[...] = di * jnp.dot(h, w_ref[...],
                              preferred_element_type=jnp.float32)

  return pl.pallas_call(
      body,
      out_shape=jax.ShapeDtypeStruct((n, w2.shape[1]), jnp.float32),
  )(accp, y1, dinv, b1, w2)


def _tc_final(accp, y2, dinv, b2):
  def body(a_ref, y_ref, di_ref, b_ref, o_ref):
    s = a_ref[0] + a_ref[1] + y_ref[...]
    o_ref[...] = di_ref[...] * s + b_ref[...]

  return pl.pallas_call(
      body,
      out_shape=jax.ShapeDtypeStruct(y2.shape, jnp.float32),
  )(accp, y2, dinv, b2)


def _pad_edges(src, dst, n_nodes, chunk, n_rows):
  n_edges = src.shape[0]
  n_pad = n_rows * chunk - n_edges
  pad_dst = n_nodes + (jnp.arange(n_pad, dtype=jnp.int32) % _PAD)
  pad_src = jnp.arange(n_pad, dtype=jnp.int32) % jnp.int32(n_nodes)
  src_p = jnp.concatenate([src, pad_src])
  dst_p = jnp.concatenate([dst, pad_dst])
  return src_p.reshape(-1, chunk), dst_p.reshape(-1, chunk)


def kernel(x, edge_index, W1, b1, W2, b2):
  n = x.shape[0]
  n_edges = edge_index.shape[1]
  src = edge_index[0].astype(jnp.int32)
  dst = edge_index[1].astype(jnp.int32)

  n_rows = -(-n_edges // _CHUNK)
  n_rows += (-n_rows) % _GRP
  srcp, dstp = _pad_edges(src, dst, n, _CHUNK, n_rows)

  degp = _deg_count(dstp, n)
  dinv, y1 = _tc_dinv_scale(degp, x, W1)
  acc1 = _edge_agg(srcp, dstp, y1, n, 4, 8)
  y2 = _tc_mid(acc1, y1, dinv, b1.reshape(1, -1), W2)
  acc2 = _edge_agg(srcp, dstp, y2, n, 2, 8)
  out = _tc_final(acc2, y2, dinv, b2.reshape(1, -1))
  return out


# speedup vs baseline: 1.3857x; 1.0785x over previous
import functools

import jax
import jax.numpy as jnp
from jax import lax
from jax.experimental import pallas as pl
from jax.experimental.pallas import tpu as pltpu
from jax.experimental.pallas import tpu_sc as plsc

_NC = 2
_NS = 16
_NW = _NC * _NS
_CHUNK = 128
_GRP = 8
_DEG_G = 8
_DEG_W = 16
_RBLK = 80
_PAD = 128


def _deg_count(dst2d, n_nodes):
  n_groups = dst2d.shape[0] // _DEG_G
  iters = (n_groups + _NW - 1) // _NW
  n_blk = n_nodes // _RBLK
  blk_iters = (n_blk + _NS - 1) // _NS
  mesh = plsc.VectorSubcoreMesh(core_axis_name="c", subcore_axis_name="s")
  ones = jnp.ones((_CHUNK, _DEG_W), jnp.float32)
  zeros = jnp.zeros((_RBLK, _DEG_W), jnp.float32)

  @functools.partial(
      pl.kernel,
      out_type=jax.ShapeDtypeStruct((_NC, n_nodes, _DEG_W), jnp.float32),
      mesh=mesh,
      compiler_params=pltpu.CompilerParams(use_tc_tiling_on_sc=False),
      scratch_types=[
          pltpu.VMEM((_DEG_G, _CHUNK), jnp.int32),
          pltpu.VMEM((_CHUNK, _DEG_W), jnp.float32),
          pltpu.VMEM((_RBLK, _DEG_W), jnp.float32),
          pltpu.VMEM_SHARED((n_nodes + _PAD, _DEG_W), jnp.float32),
          pltpu.SemaphoreType.DMA,
      ],
  )
  def k(dst_h, ones_h, z_h, out_h, dst_v, ones_v, stage_v, acc, sem):
    cid = lax.axis_index("c")
    sid = lax.axis_index("s")
    wid = sid * _NC + cid
    pltpu.sync_copy(ones_h, ones_v)

    def zbody(i, carry):
      b = sid + i * _NS

      @pl.when(b < n_blk)
      def _():
        pltpu.sync_copy(z_h, acc.at[pl.ds(b * _RBLK, _RBLK), :])

      return carry

    lax.fori_loop(0, blk_iters, zbody, 0)
    plsc.subcore_barrier()

    def body(i, carry):
      g = wid + i * _NW

      @pl.when(g < n_groups)
      def _():
        pltpu.sync_copy(dst_h.at[pl.ds(g * _DEG_G, _DEG_G), :], dst_v)
        descs = [
            pltpu.async_copy(ones_v, acc.at[dst_v.at[j]], sem, add=True)
            for j in range(_DEG_G)
        ]
        for d_ in descs:
          d_.wait()

      return carry

    lax.fori_loop(0, iters, body, 0)
    plsc.subcore_barrier()

    def obody(i, carry):
      b = sid + i * _NS

      @pl.when(b < n_blk)
      def _():
        pltpu.sync_copy(acc.at[pl.ds(b * _RBLK, _RBLK), :], stage_v)
        pltpu.sync_copy(stage_v, out_h.at[cid, pl.ds(b * _RBLK, _RBLK), :])

      return carry

    lax.fori_loop(0, blk_iters, obody, 0)

  return k(dst2d, ones, zeros)


def _edge_agg(src2d, dst2d, table, n_nodes, nbuf, grp):
  d = table.shape[1]
  n_groups = src2d.shape[0] // grp
  iters = (n_groups + _NW - 1) // _NW
  n_blk = n_nodes // _RBLK
  blk_iters = (n_blk + _NS - 1) // _NS
  mesh = plsc.VectorSubcoreMesh(core_axis_name="c", subcore_axis_name="s")
  zeros = jnp.zeros((_RBLK, d), jnp.float32)

  @functools.partial(
      pl.kernel,
      out_type=jax.ShapeDtypeStruct((_NC, n_nodes, d), jnp.float32),
      mesh=mesh,
      compiler_params=pltpu.CompilerParams(use_tc_tiling_on_sc=False),
      scratch_types=(
          [pltpu.VMEM((grp, _CHUNK), jnp.int32),
           pltpu.VMEM((grp, _CHUNK), jnp.int32)]
          + [pltpu.VMEM((_CHUNK, d), jnp.float32) for _ in range(nbuf)]
          + [pltpu.VMEM((_RBLK, d), jnp.float32),
             pltpu.VMEM_SHARED((n_nodes + _PAD, d), jnp.float32)]
          + [pltpu.SemaphoreType.DMA for _ in range(2 * nbuf)]
      ),
  )
  def k(src_h, dst_h, tab_h, z_h, out_h, src_v, dst_v, *rest):
    rows = rest[:nbuf]
    stage_v = rest[nbuf]
    acc = rest[nbuf + 1]
    sems = rest[nbuf + 2:2 * nbuf + 2]
    ssems = rest[2 * nbuf + 2:]
    cid = lax.axis_index("c")
    sid = lax.axis_index("s")
    wid = sid * _NC + cid

    def zbody(i, carry):
      b = sid + i * _NS

      @pl.when(b < n_blk)
      def _():
        pltpu.sync_copy(z_h, acc.at[pl.ds(b * _RBLK, _RBLK), :])

      return carry

    lax.fori_loop(0, blk_iters, zbody, 0)
    plsc.subcore_barrier()

    def body(i, carry):
      g = wid + i * _NW

      @pl.when(g < n_groups)
      def _():
        pltpu.sync_copy(src_h.at[pl.ds(g * grp, grp), :], src_v)
        pltpu.sync_copy(dst_h.at[pl.ds(g * grp, grp), :], dst_v)
        gd = [None] * grp
        sd = [None] * grp
        for p in range(nbuf - 1):
          gd[p] = pltpu.async_copy(
              tab_h.at[src_v.at[p]], rows[p % nbuf], sems[p % nbuf])
        for j in range(grp):
          if j >= 1:
            sd[j - 1].wait()
          nx = j + nbuf - 1
          if nx < grp:
            gd[nx] = pltpu.async_copy(
                tab_h.at[src_v.at[nx]], rows[nx % nbuf], sems[nx % nbuf])
          gd[j].wait()
          sd[j] = pltpu.async_copy(rows[j % nbuf], acc.at[dst_v.at[j]],
                                   ssems[j % nbuf], add=True)
        sd[grp - 1].wait()

      return carry

    lax.fori_loop(0, iters, body, 0)
    plsc.subcore_barrier()

    def obody(i, carry):
      b = sid + i * _NS

      @pl.when(b < n_blk)
      def _():
        pltpu.sync_copy(acc.at[pl.ds(b * _RBLK, _RBLK), :], stage_v)
        pltpu.sync_copy(stage_v, out_h.at[cid, pl.ds(b * _RBLK, _RBLK), :])

      return carry

    lax.fori_loop(0, blk_iters, obody, 0)

  return k(src2d, dst2d, table, zeros)


def _tc_dinv_scale(degp, x, w1):
  n = x.shape[0]

  def body(d_ref, x_ref, w_ref, dinv_ref, y_ref):
    dsum = d_ref[0] + d_ref[1]
    deg = dsum[:, 0:1] + 1.0
    dinv = lax.rsqrt(deg)
    dinv_ref[...] = dinv
    xw = jnp.dot(x_ref[...], w_ref[...], preferred_element_type=jnp.float32)
    y_ref[...] = xw * dinv

  return pl.pallas_call(
      body,
      out_shape=(
          jax.ShapeDtypeStruct((n, 1), jnp.float32),
          jax.ShapeDtypeStruct((n, w1.shape[1]), jnp.float32),
      ),
  )(degp, x, w1)


def _tc_mid(accp, y1, dinv, b1, w2):
  n = y1.shape[0]

  def body(a_ref, y_ref, di_ref, b_ref, w_ref, o_ref):
    di = di_ref[...]
    s = a_ref[0] + a_ref[1] + y_ref[...]
    h = jnp.maximum(di * s + b_ref[...], 0.0)
    o_ref[...] = di * jnp.dot(h, w_ref[...],
                              preferred_element_type=jnp.float32)

  return pl.pallas_call(
      body,
      out_shape=jax.ShapeDtypeStruct((n, w2.shape[1]), jnp.float32),
  )(accp, y1, dinv, b1, w2)


def _tc_final(accp, y2, dinv, b2):
  def body(a_ref, y_ref, di_ref, b_ref, o_ref):
    s = a_ref[0] + a_ref[1] + y_ref[...]
    o_ref[...] = di_ref[...] * s + b_ref[...]

  return pl.pallas_call(
      body,
      out_shape=jax.ShapeDtypeStruct(y2.shape, jnp.float32),
  )(accp, y2, dinv, b2)


def _pad_edges(src, dst, n_nodes, chunk, n_rows):
  n_edges = src.shape[0]
  n_pad = n_rows * chunk - n_edges
  pad_dst = n_nodes + (jnp.arange(n_pad, dtype=jnp.int32) % _PAD)
  pad_src = jnp.arange(n_pad, dtype=jnp.int32) % jnp.int32(n_nodes)
  src_p = jnp.concatenate([src, pad_src])
  dst_p = jnp.concatenate([dst, pad_dst])
  return src_p.reshape(-1, chunk), dst_p.reshape(-1, chunk)


def kernel(x, edge_index, W1, b1, W2, b2):
  n = x.shape[0]
  n_edges = edge_index.shape[1]
  src = edge_index[0].astype(jnp.int32)
  dst = edge_index[1].astype(jnp.int32)

  n_rows = -(-n_edges // _CHUNK)
  n_rows += (-n_rows) % 16
  srcp, dstp = _pad_edges(src, dst, n, _CHUNK, n_rows)

  degp = _deg_count(dstp, n)
  dinv, y1 = _tc_dinv_scale(degp, x, W1)
  acc1 = _edge_agg(srcp, dstp, y1, n, 4, 16)
  y2 = _tc_mid(acc1, y1, dinv, b1.reshape(1, -1), W2)
  acc2 = _edge_agg(srcp, dstp, y2, n, 2, 16)
  out = _tc_final(acc2, y2, dinv, b2.reshape(1, -1))
  return out


# speedup vs baseline: 1.4002x; 1.0105x over previous
import functools

import jax
import jax.numpy as jnp
from jax import lax
from jax.experimental import pallas as pl
from jax.experimental.pallas import tpu as pltpu
from jax.experimental.pallas import tpu_sc as plsc

_NC = 2
_NS = 16
_NW = _NC * _NS
_CHUNK = 128
_GRP = 8
_DEG_G = 16
_DEG_W = 16
_RBLK = 80
_PAD = 128


def _deg_count(dst2d, n_nodes):
  n_groups = dst2d.shape[0] // _DEG_G
  iters = (n_groups + _NW - 1) // _NW
  n_blk = n_nodes // _RBLK
  blk_iters = (n_blk + _NS - 1) // _NS
  mesh = plsc.VectorSubcoreMesh(core_axis_name="c", subcore_axis_name="s")
  ones = jnp.ones((_CHUNK, _DEG_W), jnp.float32)
  zeros = jnp.zeros((_RBLK, _DEG_W), jnp.float32)

  @functools.partial(
      pl.kernel,
      out_type=jax.ShapeDtypeStruct((_NC, n_nodes, _DEG_W), jnp.float32),
      mesh=mesh,
      compiler_params=pltpu.CompilerParams(use_tc_tiling_on_sc=False),
      scratch_types=[
          pltpu.VMEM((_DEG_G, _CHUNK), jnp.int32),
          pltpu.VMEM((_CHUNK, _DEG_W), jnp.float32),
          pltpu.VMEM((_RBLK, _DEG_W), jnp.float32),
          pltpu.VMEM_SHARED((n_nodes + _PAD, _DEG_W), jnp.float32),
          pltpu.SemaphoreType.DMA,
      ],
  )
  def k(dst_h, ones_h, z_h, out_h, dst_v, ones_v, stage_v, acc, sem):
    cid = lax.axis_index("c")
    sid = lax.axis_index("s")
    wid = sid * _NC + cid
    pltpu.sync_copy(ones_h, ones_v)

    def zbody(i, carry):
      b = sid + i * _NS

      @pl.when(b < n_blk)
      def _():
        pltpu.sync_copy(z_h, acc.at[pl.ds(b * _RBLK, _RBLK), :])

      return carry

    lax.fori_loop(0, blk_iters, zbody, 0)
    plsc.subcore_barrier()

    def body(i, carry):
      g = wid + i * _NW

      @pl.when(g < n_groups)
      def _():
        pltpu.sync_copy(dst_h.at[pl.ds(g * _DEG_G, _DEG_G), :], dst_v)
        descs = [
            pltpu.async_copy(ones_v, acc.at[dst_v.at[j]], sem, add=True)
            for j in range(_DEG_G)
        ]
        for d_ in descs:
          d_.wait()

      return carry

    lax.fori_loop(0, iters, body, 0)
    plsc.subcore_barrier()

    def obody(i, carry):
      b = sid + i * _NS

      @pl.when(b < n_blk)
      def _():
        pltpu.sync_copy(acc.at[pl.ds(b * _RBLK, _RBLK), :], stage_v)
        pltpu.sync_copy(stage_v, out_h.at[cid, pl.ds(b * _RBLK, _RBLK), :])

      return carry

    lax.fori_loop(0, blk_iters, obody, 0)

  return k(dst2d, ones, zeros)


def _edge_agg(src2d, dst2d, table, n_nodes, nbuf, grp):
  d = table.shape[1]
  n_groups = src2d.shape[0] // grp
  iters = (n_groups + _NW - 1) // _NW
  n_blk = n_nodes // _RBLK
  blk_iters = (n_blk + _NS - 1) // _NS
  mesh = plsc.VectorSubcoreMesh(core_axis_name="c", subcore_axis_name="s")
  zeros = jnp.zeros((_RBLK, d), jnp.float32)

  @functools.partial(
      pl.kernel,
      out_type=jax.ShapeDtypeStruct((_NC, n_nodes, d), jnp.float32),
      mesh=mesh,
      compiler_params=pltpu.CompilerParams(use_tc_tiling_on_sc=False),
      scratch_types=(
          [pltpu.VMEM((grp, _CHUNK), jnp.int32),
           pltpu.VMEM((grp, _CHUNK), jnp.int32)]
          + [pltpu.VMEM((_CHUNK, d), jnp.float32) for _ in range(nbuf)]
          + [pltpu.VMEM((_RBLK, d), jnp.float32),
             pltpu.VMEM_SHARED((n_nodes + _PAD, d), jnp.float32)]
          + [pltpu.SemaphoreType.DMA for _ in range(2 * nbuf)]
      ),
  )
  def k(src_h, dst_h, tab_h, z_h, out_h, src_v, dst_v, *rest):
    rows = rest[:nbuf]
    stage_v = rest[nbuf]
    acc = rest[nbuf + 1]
    sems = rest[nbuf + 2:2 * nbuf + 2]
    ssems = rest[2 * nbuf + 2:]
    cid = lax.axis_index("c")
    sid = lax.axis_index("s")
    wid = sid * _NC + cid

    def zbody(i, carry):
      b = sid + i * _NS

      @pl.when(b < n_blk)
      def _():
        pltpu.sync_copy(z_h, acc.at[pl.ds(b * _RBLK, _RBLK), :])

      return carry

    lax.fori_loop(0, blk_iters, zbody, 0)
    plsc.subcore_barrier()

    def body(i, carry):
      g = wid + i * _NW

      @pl.when(g < n_groups)
      def _():
        pltpu.sync_copy(src_h.at[pl.ds(g * grp, grp), :], src_v)
        pltpu.sync_copy(dst_h.at[pl.ds(g * grp, grp), :], dst_v)
        gd = [None] * grp
        sd = [None] * grp
        for p in range(nbuf - 1):
          gd[p] = pltpu.async_copy(
              tab_h.at[src_v.at[p]], rows[p % nbuf], sems[p % nbuf])
        for j in range(grp):
          if j >= 1:
            sd[j - 1].wait()
          nx = j + nbuf - 1
          if nx < grp:
            gd[nx] = pltpu.async_copy(
                tab_h.at[src_v.at[nx]], rows[nx % nbuf], sems[nx % nbuf])
          gd[j].wait()
          sd[j] = pltpu.async_copy(rows[j % nbuf], acc.at[dst_v.at[j]],
                                   ssems[j % nbuf], add=True)
        sd[grp - 1].wait()

      return carry

    lax.fori_loop(0, iters, body, 0)
    plsc.subcore_barrier()

    def obody(i, carry):
      b = sid + i * _NS

      @pl.when(b < n_blk)
      def _():
        pltpu.sync_copy(acc.at[pl.ds(b * _RBLK, _RBLK), :], stage_v)
        pltpu.sync_copy(stage_v, out_h.at[cid, pl.ds(b * _RBLK, _RBLK), :])

      return carry

    lax.fori_loop(0, blk_iters, obody, 0)

  return k(src2d, dst2d, table, zeros)


def _tc_dinv_scale(degp, x, w1):
  n = x.shape[0]

  def body(d_ref, x_ref, w_ref, dinv_ref, y_ref):
    dsum = d_ref[0] + d_ref[1]
    deg = dsum[:, 0:1] + 1.0
    dinv = lax.rsqrt(deg)
    dinv_ref[...] = dinv
    xw = jnp.dot(x_ref[...], w_ref[...], preferred_element_type=jnp.float32)
    y_ref[...] = xw * dinv

  return pl.pallas_call(
      body,
      out_shape=(
          jax.ShapeDtypeStruct((n, 1), jnp.float32),
          jax.ShapeDtypeStruct((n, w1.shape[1]), jnp.float32),
      ),
  )(degp, x, w1)


def _tc_mid(accp, y1, dinv, b1, w2):
  n = y1.shape[0]

  def body(a_ref, y_ref, di_ref, b_ref, w_ref, o_ref):
    di = di_ref[...]
    s = a_ref[0] + a_ref[1] + y_ref[...]
    h = jnp.maximum(di * s + b_ref[...], 0.0)
    o_ref[...] = di * jnp.dot(h, w_ref[...],
                              preferred_element_type=jnp.float32)

  return pl.pallas_call(
      body,
      out_shape=jax.ShapeDtypeStruct((n, w2.shape[1]), jnp.float32),
  )(accp, y1, dinv, b1, w2)


def _tc_final(accp, y2, dinv, b2):
  def body(a_ref, y_ref, di_ref, b_ref, o_ref):
    s = a_ref[0] + a_ref[1] + y_ref[...]
    o_ref[...] = di_ref[...] * s + b_ref[...]

  return pl.pallas_call(
      body,
      out_shape=jax.ShapeDtypeStruct(y2.shape, jnp.float32),
  )(accp, y2, dinv, b2)


def _pad_edges(src, dst, n_nodes, chunk, n_rows):
  n_edges = src.shape[0]
  n_pad = n_rows * chunk - n_edges
  pad_dst = n_nodes + (jnp.arange(n_pad, dtype=jnp.int32) % _PAD)
  pad_src = jnp.arange(n_pad, dtype=jnp.int32) % jnp.int32(n_nodes)
  src_p = jnp.concatenate([src, pad_src])
  dst_p = jnp.concatenate([dst, pad_dst])
  return src_p.reshape(-1, chunk), dst_p.reshape(-1, chunk)


def kernel(x, edge_index, W1, b1, W2, b2):
  n = x.shape[0]
  n_edges = edge_index.shape[1]
  src = edge_index[0].astype(jnp.int32)
  dst = edge_index[1].astype(jnp.int32)

  n_rows = -(-n_edges // _CHUNK)
  n_rows += (-n_rows) % 32
  srcp, dstp = _pad_edges(src, dst, n, _CHUNK, n_rows)

  degp = _deg_count(dstp, n)
  dinv, y1 = _tc_dinv_scale(degp, x, W1)
  acc1 = _edge_agg(srcp, dstp, y1, n, 4, 32)
  y2 = _tc_mid(acc1, y1, dinv, b1.reshape(1, -1), W2)
  acc2 = _edge_agg(srcp, dstp, y2, n, 2, 16)
  out = _tc_final(acc2, y2, dinv, b2.reshape(1, -1))
  return out
